# baseline (speedup 1.0000x reference)
import functools

import jax
import jax.numpy as jnp
from jax import lax
from jax.experimental import pallas as pl
from jax.experimental.pallas import tpu as pltpu
from jax.experimental.pallas import tpu_sc as plsc

B_ROWS = 4096
SEQ = 200
D = 64
SCALE = 8.0

B_TOTAL = B_ROWS * SEQ
NC = 2
NS = 16
NW = NC * NS
B_PER_W = B_TOTAL // NW

IDX_MINOR = 128
CHUNK = 1024
N_GATH = CHUNK // IDX_MINOR
N_CHUNKS = B_PER_W // CHUNK

_mesh = plsc.VectorSubcoreMesh(core_axis_name="c", subcore_axis_name="s")


@functools.partial(
    pl.kernel,
    mesh=_mesh,
    out_type=jax.ShapeDtypeStruct((B_TOTAL, D), jnp.float32),
    scratch_types=[
        pltpu.VMEM((N_GATH, IDX_MINOR), jnp.int32),
        pltpu.VMEM((CHUNK, D), jnp.float32),
        pltpu.SemaphoreType.DMA,
    ],
    compiler_params=pltpu.CompilerParams(use_tc_tiling_on_sc=False),
)
def _embed(idx_hbm, tab_hbm, out_hbm, idx_v, rows_v, sem):
    wid = lax.axis_index("s") * NC + lax.axis_index("c")
    base = wid * B_PER_W

    def chunk_body(g, carry):
        off = base + g * CHUNK
        idx_off = pl.multiple_of(off // IDX_MINOR, 8)
        pltpu.sync_copy(idx_hbm.at[pl.ds(idx_off, N_GATH)], idx_v)
        for k in range(N_GATH):
            pltpu.async_copy(
                tab_hbm.at[idx_v.at[k]],
                rows_v.at[pl.ds(k * IDX_MINOR, IDX_MINOR)],
                sem,
            )
        for k in range(N_GATH):
            pltpu.make_async_copy(
                tab_hbm.at[idx_v.at[k]],
                rows_v.at[pl.ds(k * IDX_MINOR, IDX_MINOR)],
                sem,
            ).wait()
        def scale_row(r, c):
            for j in range(D // 16):
                sl = pl.ds(j * 16, 16)
                rows_v[r, sl] = rows_v[r, sl] * SCALE
            return c

        lax.fori_loop(0, CHUNK, scale_row, 0, unroll=2)
        pltpu.sync_copy(rows_v, out_hbm.at[pl.ds(off, CHUNK)])
        return carry

    lax.fori_loop(0, N_CHUNKS, chunk_body, 0)


def kernel(x, lut):
    idx2 = x.reshape(B_TOTAL // IDX_MINOR, IDX_MINOR).astype(jnp.int32)
    out = _embed(idx2, lut)
    return out.reshape(B_ROWS, SEQ, D)
